# Initial kernel scaffold; baseline (speedup 1.0000x reference)
import jax
import jax.numpy as jnp
from jax.experimental import pallas as pl


def kernel(input_features, edge_index, W1, b1, W2, b2, W3, b3):
    raise NotImplementedError("write your pallas kernel here")


# speedup vs baseline: 1.7522x; 1.7522x over previous
import functools

import jax
import jax.numpy as jnp
from jax import lax
from jax.experimental import pallas as pl
from jax.experimental.pallas import tpu as pltpu
from jax.experimental.pallas import tpu_sc as plsc

N = 10000
E = 320000
D_H = 128
D_OUT = 40

NC = 2
NS = 16
NW = NC * NS
C = 128
NCH = 80
EP = NW * NCH * C
NP = 10240
DUMMY = N
RPT = NP // NS
DW = 8

_MESH = plsc.VectorSubcoreMesh(core_axis_name="c", subcore_axis_name="s")


@functools.partial(
    pl.kernel,
    out_type=jax.ShapeDtypeStruct((NC, NP, D_H), jnp.float32),
    mesh=_MESH,
    scratch_types=[
        pltpu.VMEM((C, D_H), jnp.float32),
        pltpu.VMEM((C, D_H), jnp.float32),
        pltpu.VMEM((8, C), jnp.int32),
        pltpu.VMEM_SHARED((NP, D_H), jnp.float32),
        pltpu.SemaphoreType.DMA,
        pltpu.SemaphoreType.DMA,
        pltpu.SemaphoreType.DMA,
        pltpu.SemaphoreType.DMA,
    ],
)
def _sc_agg(hw_hbm, srcp_hbm, dstp_hbm, out_hbm,
            buf0, buf1, iring, acc, semg0, semg1, semi0, semi1):
    c = lax.axis_index("c")
    s = lax.axis_index("s")
    w = c * NS + s
    zero16 = jnp.zeros((16,), jnp.float32)

    @pl.loop(0, C)
    def _(r):
        for t in range(D_H // 16):
            buf0[r, pl.ds(t * 16, 16)] = zero16

    base = s * RPT
    for k in range(RPT // C):
        pltpu.sync_copy(buf0, acc.at[pl.ds(base + k * C, C)])
    plsc.subcore_barrier()

    @pl.loop(0, NCH)
    def _(j):
        pltpu.sync_copy(srcp_hbm.at[w, j], iring.at[0])
        pltpu.sync_copy(dstp_hbm.at[w, j], iring.at[1])
        pltpu.async_copy(hw_hbm.at[iring.at[0]], buf0, semg0).wait()
        pltpu.sync_copy(buf0, acc.at[iring.at[1]], add=True)

    plsc.subcore_barrier()
    for k in range(RPT // C):
        pltpu.sync_copy(acc.at[pl.ds(base + k * C, C)], buf0)
        pltpu.sync_copy(buf0, out_hbm.at[c, pl.ds(base + k * C, C)])


def _pre_body(x_ref, d_ref, w_ref, o_ref):
    no = lax.rsqrt(jnp.maximum(d_ref[0, 0] + d_ref[1, 0], 1.0))
    o_ref[...] = jnp.dot(x_ref[...] * no, w_ref[...],
                         preferred_element_type=jnp.float32)


def _tc_pre(x_pad, degs4, w):
    rb = 256
    return pl.pallas_call(
        _pre_body,
        grid=(NP // rb,),
        in_specs=[
            pl.BlockSpec((rb, D_H), lambda i: (i, 0)),
            pl.BlockSpec((NC, 2, rb, 1), lambda i: (0, 0, i, 0)),
            pl.BlockSpec((D_H, D_H), lambda i: (0, 0)),
        ],
        out_specs=pl.BlockSpec((rb, D_H), lambda i: (i, 0)),
        out_shape=jax.ShapeDtypeStruct((NP, D_H), jnp.float32),
    )(x_pad, degs4, w)


def _postpre_body(p_ref, d_ref, b_ref, w_ref, o_ref):
    ni = lax.rsqrt(jnp.maximum(d_ref[0, 1] + d_ref[1, 1], 1.0))
    h = jnp.maximum((p_ref[0] + p_ref[1]) * ni + b_ref[...], 0.0)
    no = lax.rsqrt(jnp.maximum(d_ref[0, 0] + d_ref[1, 0], 1.0))
    o_ref[...] = jnp.dot(h * no, w_ref[...],
                         preferred_element_type=jnp.float32)


def _tc_postpre(parts, degs4, b, w):
    rb = 256
    return pl.pallas_call(
        _postpre_body,
        grid=(NP // rb,),
        in_specs=[
            pl.BlockSpec((NC, rb, D_H), lambda i: (0, i, 0)),
            pl.BlockSpec((NC, 2, rb, 1), lambda i: (0, 0, i, 0)),
            pl.BlockSpec((1, D_H), lambda i: (0, 0)),
            pl.BlockSpec((D_H, D_H), lambda i: (0, 0)),
        ],
        out_specs=pl.BlockSpec((rb, D_H), lambda i: (i, 0)),
        out_shape=jax.ShapeDtypeStruct((NP, D_H), jnp.float32),
    )(parts, degs4, b, w)


def _final_body(p_ref, d_ref, b_ref, o_ref):
    ni = lax.rsqrt(jnp.maximum(d_ref[0, 1] + d_ref[1, 1], 1.0))
    z = (p_ref[0] + p_ref[1]) * ni + b_ref[...]
    col = lax.broadcasted_iota(jnp.int32, z.shape, 1)
    z = jnp.where(col < D_OUT, z, -1e30)
    m = jnp.max(z, axis=-1, keepdims=True)
    lse = jnp.log(jnp.sum(jnp.exp(z - m), axis=-1, keepdims=True)) + m
    o_ref[...] = (z - lse)[:, :D_OUT]


def _tc_final(parts, degs4, b3p):
    rb = 200
    return pl.pallas_call(
        _final_body,
        grid=(N // rb,),
        in_specs=[
            pl.BlockSpec((NC, rb, D_H), lambda i: (0, i, 0)),
            pl.BlockSpec((NC, 2, rb, 1), lambda i: (0, 0, i, 0)),
            pl.BlockSpec((1, D_H), lambda i: (0, 0)),
        ],
        out_specs=pl.BlockSpec((rb, D_OUT), lambda i: (i, 0)),
        out_shape=jax.ShapeDtypeStruct((N, D_OUT), jnp.float32),
    )(parts, degs4, b3p)


def kernel(input_features, edge_index, W1, b1, W2, b2, W3, b3):
    pad_idx = jnp.full((EP - E,), DUMMY, jnp.int32)
    src_p = jnp.concatenate([edge_index[0], pad_idx]).reshape(NW, NCH, C)
    dst_p = jnp.concatenate([edge_index[1], pad_idx]).reshape(NW, NCH, C)
    x_pad = jnp.concatenate(
        [input_features, jnp.zeros((NP - N, D_H), jnp.float32)], axis=0)
    w3p = jnp.pad(W3, ((0, 0), (0, D_H - D_OUT)))
    b3p = jnp.pad(b3, (0, D_H - D_OUT)).reshape(1, D_H)

    ones_np = jnp.ones((NP, D_H), jnp.float32)
    po = _sc_agg(ones_np, dst_p, src_p)
    pi = _sc_agg(ones_np, src_p, dst_p)
    degs4 = jnp.stack([po[..., :1], pi[..., :1]], axis=1)

    y1 = _tc_pre(x_pad, degs4, W1)
    p1 = _sc_agg(y1, src_p, dst_p)
    y2 = _tc_postpre(p1, degs4, b1.reshape(1, D_H), W2)
    p2 = _sc_agg(y2, src_p, dst_p)
    y3 = _tc_postpre(p2, degs4, b2.reshape(1, D_H), w3p)
    p3 = _sc_agg(y3, src_p, dst_p)
    return _tc_final(p3, degs4, b3p)


# speedup vs baseline: 2.1126x; 1.2057x over previous
import functools

import jax
import jax.numpy as jnp
from jax import lax
from jax.experimental import pallas as pl
from jax.experimental.pallas import tpu as pltpu
from jax.experimental.pallas import tpu_sc as plsc

N = 10000
E = 320000
D_H = 128
D_OUT = 40

NC = 2
NS = 16
NW = NC * NS
C = 128
NCH = 80
EP = NW * NCH * C
NP = 10240
DUMMY = N
RPT = NP // NS
DW = 8

_MESH = plsc.VectorSubcoreMesh(core_axis_name="c", subcore_axis_name="s")


@functools.partial(
    pl.kernel,
    out_type=jax.ShapeDtypeStruct((NC, NP, D_H), jnp.float32),
    mesh=_MESH,
    scratch_types=[
        pltpu.VMEM((C, D_H), jnp.float32),
        pltpu.VMEM((C, D_H), jnp.float32),
        pltpu.VMEM((8, C), jnp.int32),
        pltpu.VMEM_SHARED((NP, D_H), jnp.float32),
        pltpu.SemaphoreType.DMA,
        pltpu.SemaphoreType.DMA,
        pltpu.SemaphoreType.DMA,
        pltpu.SemaphoreType.DMA,
        pltpu.SemaphoreType.DMA,
        pltpu.SemaphoreType.DMA,
    ],
)
def _sc_agg(hw_hbm, srcp_hbm, dstp_hbm, out_hbm,
            buf0, buf1, iring, acc,
            semg0, semg1, semi0, semi1, sems0, sems1):
    c = lax.axis_index("c")
    s = lax.axis_index("s")
    w = c * NS + s
    zero16 = jnp.zeros((16,), jnp.float32)

    @pl.loop(0, C)
    def _(r):
        for t in range(D_H // 16):
            buf0[r, pl.ds(t * 16, 16)] = zero16

    base = s * RPT
    for k in range(RPT // C):
        pltpu.sync_copy(buf0, acc.at[pl.ds(base + k * C, C)])
    plsc.subcore_barrier()

    bufs = (buf0, buf1)
    semg = (semg0, semg1)
    semi = (semi0, semi1)
    sems = (sems0, sems1)

    def idx_rows(j):
        p = (j % 4) * 2
        return iring.at[p], iring.at[p + 1]

    def issue_idx(j):
        sr, dr = idx_rows(j)
        sem = semi[j % 2]
        return (pltpu.async_copy(srcp_hbm.at[w, j], sr, sem),
                pltpu.async_copy(dstp_hbm.at[w, j], dr, sem))

    d_idx = [None] * NCH
    d_g = [None] * NCH
    d_s = [None] * NCH

    for j in range(min(3, NCH)):
        d_idx[j] = issue_idx(j)
    d_idx[0][0].wait()
    d_idx[0][1].wait()
    d_g[0] = pltpu.async_copy(hw_hbm.at[idx_rows(0)[0]], bufs[0], semg[0])

    for j in range(NCH):
        b = j % 2
        d_g[j].wait()
        d_s[j] = pltpu.async_copy(
            bufs[b], acc.at[idx_rows(j)[1]], sems[b], add=True)
        if j + 1 < NCH:
            d_idx[j + 1][0].wait()
            d_idx[j + 1][1].wait()
            if j >= 1:
                d_s[j - 1].wait()
            d_g[j + 1] = pltpu.async_copy(
                hw_hbm.at[idx_rows(j + 1)[0]], bufs[1 - b], semg[1 - b])
        if j + 3 < NCH:
            d_idx[j + 3] = issue_idx(j + 3)
    d_s[NCH - 2].wait()
    d_s[NCH - 1].wait()

    plsc.subcore_barrier()
    for k in range(RPT // C):
        pltpu.sync_copy(acc.at[pl.ds(base + k * C, C)], bufs[k % 2])
        pltpu.sync_copy(bufs[k % 2], out_hbm.at[c, pl.ds(base + k * C, C)])


def _pre_body(x_ref, d_ref, w_ref, o_ref):
    no = lax.rsqrt(jnp.maximum(d_ref[0, 0] + d_ref[1, 0], 1.0))
    o_ref[...] = jnp.dot(x_ref[...] * no, w_ref[...],
                         preferred_element_type=jnp.float32)


def _tc_pre(x_pad, degs4, w):
    rb = 256
    return pl.pallas_call(
        _pre_body,
        grid=(NP // rb,),
        in_specs=[
            pl.BlockSpec((rb, D_H), lambda i: (i, 0)),
            pl.BlockSpec((NC, 2, rb, 1), lambda i: (0, 0, i, 0)),
            pl.BlockSpec((D_H, D_H), lambda i: (0, 0)),
        ],
        out_specs=pl.BlockSpec((rb, D_H), lambda i: (i, 0)),
        out_shape=jax.ShapeDtypeStruct((NP, D_H), jnp.float32),
    )(x_pad, degs4, w)


def _postpre_body(p_ref, d_ref, b_ref, w_ref, o_ref):
    ni = lax.rsqrt(jnp.maximum(d_ref[0, 1] + d_ref[1, 1], 1.0))
    h = jnp.maximum((p_ref[0] + p_ref[1]) * ni + b_ref[...], 0.0)
    no = lax.rsqrt(jnp.maximum(d_ref[0, 0] + d_ref[1, 0], 1.0))
    o_ref[...] = jnp.dot(h * no, w_ref[...],
                         preferred_element_type=jnp.float32)


def _tc_postpre(parts, degs4, b, w):
    rb = 256
    return pl.pallas_call(
        _postpre_body,
        grid=(NP // rb,),
        in_specs=[
            pl.BlockSpec((NC, rb, D_H), lambda i: (0, i, 0)),
            pl.BlockSpec((NC, 2, rb, 1), lambda i: (0, 0, i, 0)),
            pl.BlockSpec((1, D_H), lambda i: (0, 0)),
            pl.BlockSpec((D_H, D_H), lambda i: (0, 0)),
        ],
        out_specs=pl.BlockSpec((rb, D_H), lambda i: (i, 0)),
        out_shape=jax.ShapeDtypeStruct((NP, D_H), jnp.float32),
    )(parts, degs4, b, w)


def _final_body(p_ref, d_ref, b_ref, o_ref):
    ni = lax.rsqrt(jnp.maximum(d_ref[0, 1] + d_ref[1, 1], 1.0))
    z = (p_ref[0] + p_ref[1]) * ni + b_ref[...]
    col = lax.broadcasted_iota(jnp.int32, z.shape, 1)
    z = jnp.where(col < D_OUT, z, -1e30)
    m = jnp.max(z, axis=-1, keepdims=True)
    lse = jnp.log(jnp.sum(jnp.exp(z - m), axis=-1, keepdims=True)) + m
    o_ref[...] = (z - lse)[:, :D_OUT]


def _tc_final(parts, degs4, b3p):
    rb = 200
    return pl.pallas_call(
        _final_body,
        grid=(N // rb,),
        in_specs=[
            pl.BlockSpec((NC, rb, D_H), lambda i: (0, i, 0)),
            pl.BlockSpec((NC, 2, rb, 1), lambda i: (0, 0, i, 0)),
            pl.BlockSpec((1, D_H), lambda i: (0, 0)),
        ],
        out_specs=pl.BlockSpec((rb, D_OUT), lambda i: (i, 0)),
        out_shape=jax.ShapeDtypeStruct((N, D_OUT), jnp.float32),
    )(parts, degs4, b3p)


def kernel(input_features, edge_index, W1, b1, W2, b2, W3, b3):
    pad_idx = jnp.full((EP - E,), DUMMY, jnp.int32)
    src_p = jnp.concatenate([edge_index[0], pad_idx]).reshape(NW, NCH, C)
    dst_p = jnp.concatenate([edge_index[1], pad_idx]).reshape(NW, NCH, C)
    x_pad = jnp.concatenate(
        [input_features, jnp.zeros((NP - N, D_H), jnp.float32)], axis=0)
    w3p = jnp.pad(W3, ((0, 0), (0, D_H - D_OUT)))
    b3p = jnp.pad(b3, (0, D_H - D_OUT)).reshape(1, D_H)

    ones_np = jnp.ones((NP, D_H), jnp.float32)
    po = _sc_agg(ones_np, dst_p, src_p)
    pi = _sc_agg(ones_np, src_p, dst_p)
    degs4 = jnp.stack([po[..., :1], pi[..., :1]], axis=1)

    y1 = _tc_pre(x_pad, degs4, W1)
    p1 = _sc_agg(y1, src_p, dst_p)
    y2 = _tc_postpre(p1, degs4, b1.reshape(1, D_H), W2)
    p2 = _sc_agg(y2, src_p, dst_p)
    y3 = _tc_postpre(p2, degs4, b2.reshape(1, D_H), w3p)
    p3 = _sc_agg(y3, src_p, dst_p)
    return _tc_final(p3, degs4, b3p)


# speedup vs baseline: 2.7845x; 1.3180x over previous
import functools

import jax
import jax.numpy as jnp
from jax import lax
from jax.experimental import pallas as pl
from jax.experimental.pallas import tpu as pltpu
from jax.experimental.pallas import tpu_sc as plsc

N = 10000
E = 320000
D_H = 128
D_OUT = 40

NC = 2
NS = 16
NW = NC * NS
C = 128
NCH = 80
EP = NW * NCH * C
NP = 10240
DUMMY = N
RPT = NP // NS
DW = 8

_MESH = plsc.VectorSubcoreMesh(core_axis_name="c", subcore_axis_name="s")


ACC_R = 10112
APT = ACC_R // NS
_CHUNKS = (C, C, C, C, APT - 4 * C)


@functools.partial(
    pl.kernel,
    out_type=jax.ShapeDtypeStruct((NC, NP, D_H), jnp.float32),
    mesh=_MESH,
    scratch_types=[
        pltpu.VMEM((C, D_H), jnp.float32),
        pltpu.VMEM((C, D_H), jnp.float32),
        pltpu.VMEM((C, D_H), jnp.float32),
        pltpu.VMEM((8, C), jnp.int32),
        pltpu.VMEM_SHARED((ACC_R, D_H), jnp.float32),
        pltpu.SemaphoreType.DMA,
        pltpu.SemaphoreType.DMA,
        pltpu.SemaphoreType.DMA,
        pltpu.SemaphoreType.DMA,
        pltpu.SemaphoreType.DMA,
        pltpu.SemaphoreType.DMA,
        pltpu.SemaphoreType.DMA,
    ],
)
def _sc_agg(hw_hbm, srcp_hbm, dstp_hbm, out_hbm,
            buf0, buf1, buf2, iring, acc,
            semg0, semg1, semg2, semi0, semi1, sems0, sems1):
    c = lax.axis_index("c")
    s = lax.axis_index("s")
    w = c * NS + s
    zero16 = jnp.zeros((16,), jnp.float32)

    @pl.loop(0, C)
    def _(r):
        for t in range(D_H // 16):
            buf0[r, pl.ds(t * 16, 16)] = zero16

    base = s * APT
    off = 0
    for n in _CHUNKS:
        pltpu.sync_copy(buf0.at[pl.ds(0, n)], acc.at[pl.ds(base + off, n)])
        off += n
    plsc.subcore_barrier()

    bufs = (buf0, buf1, buf2)
    semg = (semg0, semg1, semg2)
    semi = (semi0, semi1)
    sems = (sems0, sems1)

    def idx_rows(j):
        p = (j % 4) * 2
        return iring.at[p], iring.at[p + 1]

    def issue_idx(j):
        sr, dr = idx_rows(j)
        sem = semi[j % 2]
        return (pltpu.async_copy(srcp_hbm.at[w, j], sr, sem),
                pltpu.async_copy(dstp_hbm.at[w, j], dr, sem))

    d_idx = [None] * NCH
    d_g = [None] * NCH
    d_s = [None] * NCH

    for j in range(3):
        d_idx[j] = issue_idx(j)
    d_idx[0][0].wait()
    d_idx[0][1].wait()
    d_g[0] = pltpu.async_copy(hw_hbm.at[idx_rows(0)[0]], bufs[0], semg[0])
    d_idx[1][0].wait()
    d_idx[1][1].wait()
    d_g[1] = pltpu.async_copy(hw_hbm.at[idx_rows(1)[0]], bufs[1], semg[1])

    for j in range(NCH):
        d_g[j].wait()
        d_s[j] = pltpu.async_copy(
            bufs[j % 3], acc.at[idx_rows(j)[1]], sems[j % 2], add=True)
        if j + 2 < NCH:
            d_idx[j + 2][0].wait()
            d_idx[j + 2][1].wait()
            if j >= 1:
                d_s[j - 1].wait()
            d_g[j + 2] = pltpu.async_copy(
                hw_hbm.at[idx_rows(j + 2)[0]], bufs[(j + 2) % 3],
                semg[(j + 2) % 3])
        if j + 3 < NCH:
            d_idx[j + 3] = issue_idx(j + 3)
    d_s[NCH - 3].wait()
    d_s[NCH - 2].wait()
    d_s[NCH - 1].wait()

    plsc.subcore_barrier()
    off = 0
    for i, n in enumerate(_CHUNKS):
        b = bufs[i % 2]
        pltpu.sync_copy(acc.at[pl.ds(base + off, n)], b.at[pl.ds(0, n)])
        pltpu.sync_copy(b.at[pl.ds(0, n)],
                        out_hbm.at[c, pl.ds(base + off, n)])
        off += n


@functools.partial(
    pl.kernel,
    out_type=jax.ShapeDtypeStruct((NC, 2, NP, D_H), jnp.float32),
    mesh=_MESH,
    scratch_types=[
        pltpu.VMEM((C, D_H), jnp.float32),
        pltpu.VMEM((C, D_H), jnp.float32),
        pltpu.VMEM((8, C), jnp.int32),
        pltpu.VMEM_SHARED((ACC_R, D_H), jnp.float32),
        pltpu.SemaphoreType.DMA,
        pltpu.SemaphoreType.DMA,
        pltpu.SemaphoreType.DMA,
        pltpu.SemaphoreType.DMA,
    ],
)
def _sc_degrees(ones_hbm, srcp_hbm, dstp_hbm, out_hbm,
                ones_v, bnc, iring, acc, sems0, sems1, semi0, semi1):
    c = lax.axis_index("c")
    s = lax.axis_index("s")
    w = c * NS + s
    zero16 = jnp.zeros((16,), jnp.float32)

    @pl.loop(0, C)
    def _(r):
        for t in range(D_H // 16):
            bnc[r, pl.ds(t * 16, 16)] = zero16

    pltpu.sync_copy(ones_hbm.at[pl.ds(0, C)], ones_v)
    base = s * APT
    sems = (sems0, sems1)
    for phase in range(2):
        idxp_hbm = (srcp_hbm, dstp_hbm)[phase]
        off = 0
        for n in _CHUNKS:
            pltpu.sync_copy(bnc.at[pl.ds(0, n)],
                            acc.at[pl.ds(base + off, n)])
            off += n
        plsc.subcore_barrier()

        semi = (semi0, semi1)
        d_idx = [None] * NCH
        d_s = [None] * NCH
        for j in range(min(4, NCH)):
            d_idx[j] = pltpu.async_copy(
                idxp_hbm.at[w, j], iring.at[j % 4], semi[j % 2])
        for j in range(NCH):
            d_idx[j].wait()
            if j >= 2:
                d_s[j - 2].wait()
            d_s[j] = pltpu.async_copy(
                ones_v, acc.at[iring.at[j % 4]], sems[j % 2], add=True)
            if j + 2 < NCH and j + 2 >= 4:
                d_idx[j + 2] = pltpu.async_copy(
                    idxp_hbm.at[w, j + 2], iring.at[(j + 2) % 4],
                    semi[(j + 2) % 2])
        d_s[NCH - 2].wait()
        d_s[NCH - 1].wait()

        plsc.subcore_barrier()
        off = 0
        for i, n in enumerate(_CHUNKS):
            pltpu.sync_copy(acc.at[pl.ds(base + off, n)],
                            bnc.at[pl.ds(0, n)])
            pltpu.sync_copy(bnc.at[pl.ds(0, n)],
                            out_hbm.at[c, phase, pl.ds(base + off, n)])
            off += n
        if phase == 0:
            @pl.loop(0, C)
            def _(r):
                for t in range(D_H // 16):
                    bnc[r, pl.ds(t * 16, 16)] = zero16


def _pre_body(x_ref, d_ref, w_ref, o_ref):
    no = lax.rsqrt(jnp.maximum(d_ref[0, 0] + d_ref[1, 0], 1.0))
    o_ref[...] = jnp.dot(x_ref[...] * no, w_ref[...],
                         preferred_element_type=jnp.float32)


def _tc_pre(x_pad, degs4, w):
    rb = 256
    return pl.pallas_call(
        _pre_body,
        grid=(NP // rb,),
        in_specs=[
            pl.BlockSpec((rb, D_H), lambda i: (i, 0)),
            pl.BlockSpec((NC, 2, rb, 1), lambda i: (0, 0, i, 0)),
            pl.BlockSpec((D_H, D_H), lambda i: (0, 0)),
        ],
        out_specs=pl.BlockSpec((rb, D_H), lambda i: (i, 0)),
        out_shape=jax.ShapeDtypeStruct((NP, D_H), jnp.float32),
    )(x_pad, degs4, w)


def _postpre_body(p_ref, d_ref, b_ref, w_ref, o_ref):
    ni = lax.rsqrt(jnp.maximum(d_ref[0, 1] + d_ref[1, 1], 1.0))
    h = jnp.maximum((p_ref[0] + p_ref[1]) * ni + b_ref[...], 0.0)
    no = lax.rsqrt(jnp.maximum(d_ref[0, 0] + d_ref[1, 0], 1.0))
    o_ref[...] = jnp.dot(h * no, w_ref[...],
                         preferred_element_type=jnp.float32)


def _tc_postpre(parts, degs4, b, w):
    rb = 256
    return pl.pallas_call(
        _postpre_body,
        grid=(NP // rb,),
        in_specs=[
            pl.BlockSpec((NC, rb, D_H), lambda i: (0, i, 0)),
            pl.BlockSpec((NC, 2, rb, 1), lambda i: (0, 0, i, 0)),
            pl.BlockSpec((1, D_H), lambda i: (0, 0)),
            pl.BlockSpec((D_H, D_H), lambda i: (0, 0)),
        ],
        out_specs=pl.BlockSpec((rb, D_H), lambda i: (i, 0)),
        out_shape=jax.ShapeDtypeStruct((NP, D_H), jnp.float32),
    )(parts, degs4, b, w)


def _final_body(p_ref, d_ref, b_ref, o_ref):
    ni = lax.rsqrt(jnp.maximum(d_ref[0, 1] + d_ref[1, 1], 1.0))
    z = (p_ref[0] + p_ref[1]) * ni + b_ref[...]
    col = lax.broadcasted_iota(jnp.int32, z.shape, 1)
    z = jnp.where(col < D_OUT, z, -1e30)
    m = jnp.max(z, axis=-1, keepdims=True)
    lse = jnp.log(jnp.sum(jnp.exp(z - m), axis=-1, keepdims=True)) + m
    o_ref[...] = (z - lse)[:, :D_OUT]


def _tc_final(parts, degs4, b3p):
    rb = 200
    return pl.pallas_call(
        _final_body,
        grid=(N // rb,),
        in_specs=[
            pl.BlockSpec((NC, rb, D_H), lambda i: (0, i, 0)),
            pl.BlockSpec((NC, 2, rb, 1), lambda i: (0, 0, i, 0)),
            pl.BlockSpec((1, D_H), lambda i: (0, 0)),
        ],
        out_specs=pl.BlockSpec((rb, D_OUT), lambda i: (i, 0)),
        out_shape=jax.ShapeDtypeStruct((N, D_OUT), jnp.float32),
    )(parts, degs4, b3p)


def kernel(input_features, edge_index, W1, b1, W2, b2, W3, b3):
    pad_idx = jnp.full((EP - E,), DUMMY, jnp.int32)
    src_p = jnp.concatenate([edge_index[0], pad_idx]).reshape(NW, NCH, C)
    dst_p = jnp.concatenate([edge_index[1], pad_idx]).reshape(NW, NCH, C)
    x_pad = jnp.concatenate(
        [input_features, jnp.zeros((NP - N, D_H), jnp.float32)], axis=0)
    w3p = jnp.pad(W3, ((0, 0), (0, D_H - D_OUT)))
    b3p = jnp.pad(b3, (0, D_H - D_OUT)).reshape(1, D_H)

    ones_c = jnp.ones((C, D_H), jnp.float32)
    dd = _sc_degrees(ones_c, src_p, dst_p)
    degs4 = dd[..., :1]

    y1 = _tc_pre(x_pad, degs4, W1)
    p1 = _sc_agg(y1, src_p, dst_p)
    y2 = _tc_postpre(p1, degs4, b1.reshape(1, D_H), W2)
    p2 = _sc_agg(y2, src_p, dst_p)
    y3 = _tc_postpre(p2, degs4, b2.reshape(1, D_H), w3p)
    p3 = _sc_agg(y3, src_p, dst_p)
    return _tc_final(p3, degs4, b3p)
